# baseline (speedup 1.0000x reference)
import functools

import jax
import jax.numpy as jnp
from jax import lax
from jax.experimental import pallas as pl
from jax.experimental.pallas import tpu as pltpu
from jax.experimental.pallas import tpu_sc as plsc

_NUM_CORES = 2
_NUM_SUBCORES = 16
_NUM_WORKERS = _NUM_CORES * _NUM_SUBCORES
_CHUNK = 16


def _gather_call(total, hidden, idx, table):
    b_per_w = total // _NUM_WORKERS
    n_chunks = b_per_w // _CHUNK
    mesh = plsc.VectorSubcoreMesh(core_axis_name="c", subcore_axis_name="s")

    @functools.partial(
        pl.kernel,
        mesh=mesh,
        out_type=jax.ShapeDtypeStruct((total, hidden), jnp.float32),
        scratch_types=[
            pltpu.VMEM((b_per_w,), jnp.int32),
            pltpu.VMEM((_CHUNK, hidden), jnp.float32),
            pltpu.VMEM((_CHUNK, hidden), jnp.float32),
            pltpu.VMEM((_CHUNK, hidden), jnp.float32),
            pltpu.SemaphoreType.DMA,
            pltpu.SemaphoreType.DMA,
        ],
    )
    def _gather(idx_hbm, table_hbm, out_hbm, idx_v, buf0, buf1, buf2, sem_in, sem_out):
        wid = lax.axis_index("s") * _NUM_CORES + lax.axis_index("c")
        base = wid * b_per_w
        pltpu.sync_copy(idx_hbm.at[pl.ds(base, b_per_w)], idx_v)
        bufs = (buf0, buf1, buf2)

        def gather_start(c, buf):
            pltpu.async_copy(
                table_hbm.at[idx_v.at[pl.ds(c * _CHUNK, _CHUNK)]], buf, sem_in
            )

        def gather_wait(buf):
            pltpu.make_async_copy(
                table_hbm.at[pl.ds(0, _CHUNK)], buf, sem_in
            ).wait()

        def scatter_start(c, buf):
            pltpu.async_copy(buf, out_hbm.at[pl.ds(base + c * _CHUNK, _CHUNK)], sem_out)

        def scatter_wait(c, buf):
            pltpu.make_async_copy(
                buf, out_hbm.at[pl.ds(base + c * _CHUNK, _CHUNK)], sem_out
            ).wait()

        def round_(ch, buf, prev_buf, next_buf, first=False):
            gather_wait(buf)
            scatter_start(ch, buf)
            if not first:
                scatter_wait(ch - 1, prev_buf)
            if next_buf is not None:
                gather_start(ch + 2, next_buf)

        gather_start(0, buf0)
        gather_start(1, buf1)
        round_(0, buf0, None, buf2, first=True)
        round_(1, buf1, buf0, buf0)

        n_mid = n_chunks - 5
        assert n_mid % 3 == 0

        def body(i, carry):
            ch = 2 + i * 3
            for j in range(3):
                b = (2 + j) % 3
                round_(ch + j, bufs[b], bufs[(b - 1) % 3], bufs[(b + 2) % 3])
            return carry

        lax.fori_loop(0, n_mid // 3, body, 0)

        for ch in (n_chunks - 3, n_chunks - 2, n_chunks - 1):
            nb = bufs[(ch + 2) % 3] if ch + 2 < n_chunks else None
            round_(ch, bufs[ch % 3], bufs[(ch - 1) % 3], nb)
        scatter_wait(n_chunks - 1, bufs[(n_chunks - 1) % 3])

    return _gather(idx, table)


def kernel(position_ids, weight):
    batch, seq = position_ids.shape
    vocab, hidden = weight.shape
    total = batch * seq
    idx = position_ids.reshape(total).astype(jnp.int32)
    out = _gather_call(total, hidden, idx, weight)
    return out.reshape(batch, seq, hidden)


# speedup vs baseline: 1.0014x; 1.0014x over previous
import functools

import jax
import jax.numpy as jnp
from jax import lax
from jax.experimental import pallas as pl
from jax.experimental.pallas import tpu as pltpu
from jax.experimental.pallas import tpu_sc as plsc

_NUM_CORES = 2
_NUM_SUBCORES = 16
_NUM_WORKERS = _NUM_CORES * _NUM_SUBCORES
_CHUNK = 16


def _gather_call(batch, seq, hidden, idx, table):
    total = batch * seq
    b_per_w = total // _NUM_WORKERS
    w_per_row = seq // b_per_w
    n_chunks = b_per_w // _CHUNK
    mesh = plsc.VectorSubcoreMesh(core_axis_name="c", subcore_axis_name="s")

    @functools.partial(
        pl.kernel,
        mesh=mesh,
        out_type=jax.ShapeDtypeStruct((batch, seq, hidden), jnp.float32),
        scratch_types=[
            pltpu.VMEM((b_per_w,), jnp.int32),
            pltpu.VMEM((_CHUNK, hidden), jnp.float32),
            pltpu.VMEM((_CHUNK, hidden), jnp.float32),
            pltpu.VMEM((_CHUNK, hidden), jnp.float32),
            pltpu.SemaphoreType.DMA,
            pltpu.SemaphoreType.DMA,
        ],
    )
    def _gather(idx_hbm, table_hbm, out_hbm, idx_v, buf0, buf1, buf2, sem_in, sem_out):
        wid = lax.axis_index("s") * _NUM_CORES + lax.axis_index("c")
        row = wid // w_per_row
        off = (wid % w_per_row) * b_per_w
        pltpu.sync_copy(idx_hbm.at[row, pl.ds(off, b_per_w)], idx_v)
        bufs = (buf0, buf1, buf2)

        def gather_start(c, buf):
            pltpu.async_copy(
                table_hbm.at[idx_v.at[pl.ds(c * _CHUNK, _CHUNK)]], buf, sem_in
            )

        def gather_wait(buf):
            pltpu.make_async_copy(
                table_hbm.at[pl.ds(0, _CHUNK)], buf, sem_in
            ).wait()

        def scatter_start(c, buf):
            pltpu.async_copy(
                buf, out_hbm.at[row, pl.ds(off + c * _CHUNK, _CHUNK)], sem_out
            )

        def scatter_wait(c, buf):
            pltpu.make_async_copy(
                buf, out_hbm.at[row, pl.ds(off + c * _CHUNK, _CHUNK)], sem_out
            ).wait()

        def round_(ch, buf, prev_buf, next_buf, first=False):
            gather_wait(buf)
            scatter_start(ch, buf)
            if not first:
                scatter_wait(ch - 1, prev_buf)
            if next_buf is not None:
                gather_start(ch + 2, next_buf)

        gather_start(0, buf0)
        gather_start(1, buf1)
        round_(0, buf0, None, buf2, first=True)
        round_(1, buf1, buf0, buf0)

        n_mid = n_chunks - 5
        assert n_mid % 3 == 0

        def body(i, carry):
            ch = 2 + i * 3
            for j in range(3):
                b = (2 + j) % 3
                round_(ch + j, bufs[b], bufs[(b - 1) % 3], bufs[(b + 2) % 3])
            return carry

        lax.fori_loop(0, n_mid // 3, body, 0)

        for ch in (n_chunks - 3, n_chunks - 2, n_chunks - 1):
            nb = bufs[(ch + 2) % 3] if ch + 2 < n_chunks else None
            round_(ch, bufs[ch % 3], bufs[(ch - 1) % 3], nb)
        scatter_wait(n_chunks - 1, bufs[(n_chunks - 1) % 3])

    return _gather(idx, table)


def kernel(position_ids, weight):
    batch, seq = position_ids.shape
    vocab, hidden = weight.shape
    return _gather_call(batch, seq, hidden, position_ids, weight)


# speedup vs baseline: 1.0064x; 1.0050x over previous
import functools

import jax
import jax.numpy as jnp
from jax import lax
from jax.experimental import pallas as pl
from jax.experimental.pallas import tpu as pltpu
from jax.experimental.pallas import tpu_sc as plsc

_NUM_CORES = 2
_NUM_SUBCORES = 16
_NUM_WORKERS = _NUM_CORES * _NUM_SUBCORES
_CHUNK = 16


def _gather_call(batch, seq, hidden, idx, table):
    total = batch * seq
    b_per_w = total // _NUM_WORKERS
    w_per_row = seq // b_per_w
    n_chunks = b_per_w // _CHUNK
    mesh = plsc.VectorSubcoreMesh(core_axis_name="c", subcore_axis_name="s")

    n_slots = 3

    @functools.partial(
        pl.kernel,
        mesh=mesh,
        out_type=jax.ShapeDtypeStruct((batch, seq, hidden), jnp.float32),
        scratch_types=[
            pltpu.VMEM((b_per_w,), jnp.int32),
            pltpu.VMEM((n_slots * _CHUNK, hidden), jnp.float32),
            pltpu.SemaphoreType.DMA,
            pltpu.SemaphoreType.DMA,
        ],
    )
    def _gather(idx_hbm, table_hbm, out_hbm, idx_v, ring, sem_in, sem_out):
        wid = lax.axis_index("s") * _NUM_CORES + lax.axis_index("c")
        row = wid // w_per_row
        off = (wid % w_per_row) * b_per_w
        pltpu.sync_copy(idx_hbm.at[row, pl.ds(off, b_per_w)], idx_v)

        def slot(c):
            return pl.ds(lax.rem(c, n_slots) * _CHUNK, _CHUNK)

        def body(r, carry):
            @pl.when(r >= n_slots)
            def _():
                pltpu.make_async_copy(
                    ring.at[pl.ds(0, _CHUNK)],
                    out_hbm.at[row, pl.ds(off, _CHUNK)],
                    sem_out,
                ).wait()

            @pl.when(r < n_chunks)
            def _():
                pltpu.async_copy(
                    table_hbm.at[idx_v.at[pl.ds(r * _CHUNK, _CHUNK)]],
                    ring.at[slot(r)],
                    sem_in,
                )

            @pl.when(r >= 2)
            def _():
                c = r - 2
                pltpu.make_async_copy(
                    table_hbm.at[pl.ds(0, _CHUNK)],
                    ring.at[pl.ds(0, _CHUNK)],
                    sem_in,
                ).wait()
                pltpu.async_copy(
                    ring.at[slot(c)],
                    out_hbm.at[row, pl.ds(off + c * _CHUNK, _CHUNK)],
                    sem_out,
                )

            return carry

        lax.fori_loop(0, n_chunks + 2, body, 0)
        pltpu.make_async_copy(
            ring.at[pl.ds(0, _CHUNK)],
            out_hbm.at[row, pl.ds(off, _CHUNK)],
            sem_out,
        ).wait()

    return _gather(idx, table)


def kernel(position_ids, weight):
    batch, seq = position_ids.shape
    vocab, hidden = weight.shape
    return _gather_call(batch, seq, hidden, position_ids, weight)


# speedup vs baseline: 1.0296x; 1.0231x over previous
import functools

import jax
import jax.numpy as jnp
from jax import lax
from jax.experimental import pallas as pl
from jax.experimental.pallas import tpu as pltpu
from jax.experimental.pallas import tpu_sc as plsc

_NUM_CORES = 2
_NUM_SUBCORES = 16
_NUM_WORKERS = _NUM_CORES * _NUM_SUBCORES
_CHA = 8
_CHB = 8
_SLOTS = 3


def _gather_call(batch, seq, hidden, idx, table):
    total = batch * seq
    b_per_w = total // _NUM_WORKERS
    w_per_row = seq // b_per_w
    rows_a = b_per_w // 2
    rows_b = b_per_w - rows_a
    na = rows_a // _CHA
    nb = rows_b // _CHB
    n_rounds = max(na, nb) + 2
    n_rounds += (-n_rounds) % 6
    mesh = plsc.VectorSubcoreMesh(core_axis_name="c", subcore_axis_name="s")

    @functools.partial(
        pl.kernel,
        mesh=mesh,
        out_type=jax.ShapeDtypeStruct((batch, seq, hidden), jnp.float32),
        scratch_types=[
            pltpu.VMEM((b_per_w,), jnp.int32),
            pltpu.VMEM((_SLOTS * _CHA, hidden), jnp.float32),
            pltpu.VMEM_SHARED(
                (_NUM_SUBCORES, _SLOTS * _CHB, hidden), jnp.float32
            ),
            pltpu.SemaphoreType.DMA,
            pltpu.SemaphoreType.DMA,
            pltpu.SemaphoreType.DMA,
            pltpu.SemaphoreType.DMA,
            pltpu.SemaphoreType.DMA,
            pltpu.SemaphoreType.DMA,
        ],
    )
    def _gather(idx_hbm, table_hbm, out_hbm, idx_v, ring_a, ring_b,
                sa_in, sa_out, sb0, sb1, sb2, sb_out):
        cid = lax.axis_index("c")
        sid = lax.axis_index("s")
        wid = sid * _NUM_CORES + cid
        row = wid // w_per_row
        off = (wid % w_per_row) * b_per_w
        pltpu.sync_copy(idx_hbm.at[row, pl.ds(off, b_per_w)], idx_v)
        sb = (sb0, sb1, sb2)

        def round_(i, r, p6):
            sl = p6 % _SLOTS
            @pl.when((r >= _SLOTS) & (r < na + _SLOTS))
            def _():
                pltpu.make_async_copy(
                    ring_a.at[pl.ds(0, _CHA)],
                    out_hbm.at[row, pl.ds(off, _CHA)],
                    sa_out,
                ).wait()

            @pl.when(r < na)
            def _():
                pltpu.async_copy(
                    table_hbm.at[idx_v.at[pl.ds(r * _CHA, _CHA)]],
                    ring_a.at[pl.ds(sl * _CHA, _CHA)],
                    sa_in,
                )

            @pl.when((r >= 2) & (r < na + 2))
            def _():
                c = r - 2
                pltpu.make_async_copy(
                    table_hbm.at[pl.ds(0, _CHA)],
                    ring_a.at[pl.ds(0, _CHA)],
                    sa_in,
                ).wait()
                pltpu.async_copy(
                    ring_a.at[pl.ds(((p6 - 2) % _SLOTS) * _CHA, _CHA)],
                    out_hbm.at[row, pl.ds(off + c * _CHA, _CHA)],
                    sa_out,
                )

            @pl.when((r >= _SLOTS) & (r < nb + _SLOTS))
            def _():
                pltpu.make_async_copy(
                    ring_b.at[sid, pl.ds(0, _CHB)],
                    out_hbm.at[row, pl.ds(off, _CHB)],
                    sb_out,
                ).wait()

            @pl.when((r >= 2) & (r < nb + 2))
            def _():
                c = r - 2
                pltpu.make_async_copy(
                    table_hbm.at[pl.ds(0, _CHB)],
                    ring_b.at[sid, pl.ds(0, _CHB)],
                    sb[(p6 - 2) % _SLOTS],
                ).wait()
                pltpu.async_copy(
                    ring_b.at[sid, pl.ds(((p6 - 2) % _SLOTS) * _CHB, _CHB)],
                    out_hbm.at[row, pl.ds(off + rows_a + c * _CHB, _CHB)],
                    sb_out,
                )

            @pl.when(r < nb)
            def _():
                half = 8 * (p6 % 2)
                vec = idx_v[pl.ds(rows_a + (i * 3 + p6 // 2) * 16, 16)]
                for j in range(_CHB):
                    pltpu.async_copy(
                        table_hbm.at[vec[half + j]],
                        ring_b.at[sid, sl * _CHB + j],
                        sb[sl],
                    )

        def body(i, carry):
            for p6 in range(6):
                round_(i, i * 6 + p6, p6)
            return carry

        lax.fori_loop(0, n_rounds // 6, body, 0)

    return _gather(idx, table)


def kernel(position_ids, weight):
    batch, seq = position_ids.shape
    vocab, hidden = weight.shape
    return _gather_call(batch, seq, hidden, position_ids, weight)


# speedup vs baseline: 1.0415x; 1.0115x over previous
import functools

import jax
import jax.numpy as jnp
from jax import lax
from jax.experimental import pallas as pl
from jax.experimental.pallas import tpu as pltpu
from jax.experimental.pallas import tpu_sc as plsc

_NUM_CORES = 2
_NUM_SUBCORES = 16
_NUM_WORKERS = _NUM_CORES * _NUM_SUBCORES
_CHA = 8
_CHB = 8
_SLOTS = 3


def _gather_call(batch, seq, hidden, idx, table):
    total = batch * seq
    b_per_w = total // _NUM_WORKERS
    w_per_row = seq // b_per_w
    rows_a = 3 * b_per_w // 8
    rows_b = b_per_w - rows_a
    na = rows_a // _CHA
    nb = rows_b // _CHB
    n_rounds = max(na, nb) + 2
    n_rounds += (-n_rounds) % 6
    mesh = plsc.VectorSubcoreMesh(core_axis_name="c", subcore_axis_name="s")

    @functools.partial(
        pl.kernel,
        mesh=mesh,
        out_type=jax.ShapeDtypeStruct((batch, seq, hidden), jnp.float32),
        scratch_types=[
            pltpu.VMEM((b_per_w,), jnp.int32),
            pltpu.VMEM((_SLOTS * _CHA, hidden), jnp.float32),
            pltpu.VMEM_SHARED(
                (_NUM_SUBCORES, _SLOTS * _CHB, hidden), jnp.float32
            ),
            pltpu.SemaphoreType.DMA,
            pltpu.SemaphoreType.DMA,
            pltpu.SemaphoreType.DMA,
            pltpu.SemaphoreType.DMA,
            pltpu.SemaphoreType.DMA,
            pltpu.SemaphoreType.DMA,
        ],
    )
    def _gather(idx_hbm, table_hbm, out_hbm, idx_v, ring_a, ring_b,
                sa_in, sa_out, sb0, sb1, sb2, sb_out):
        cid = lax.axis_index("c")
        sid = lax.axis_index("s")
        wid = sid * _NUM_CORES + cid
        row = wid // w_per_row
        off = (wid % w_per_row) * b_per_w
        pltpu.sync_copy(idx_hbm.at[row, pl.ds(off, b_per_w)], idx_v)
        sb = (sb0, sb1, sb2)

        def round_(i, r, p6):
            sl = p6 % _SLOTS
            @pl.when((r >= _SLOTS) & (r < na + _SLOTS))
            def _():
                pltpu.make_async_copy(
                    ring_a.at[pl.ds(0, _CHA)],
                    out_hbm.at[row, pl.ds(off, _CHA)],
                    sa_out,
                ).wait()

            @pl.when(r < na)
            def _():
                pltpu.async_copy(
                    table_hbm.at[idx_v.at[pl.ds(r * _CHA, _CHA)]],
                    ring_a.at[pl.ds(sl * _CHA, _CHA)],
                    sa_in,
                )

            @pl.when((r >= 2) & (r < na + 2))
            def _():
                c = r - 2
                pltpu.make_async_copy(
                    table_hbm.at[pl.ds(0, _CHA)],
                    ring_a.at[pl.ds(0, _CHA)],
                    sa_in,
                ).wait()
                pltpu.async_copy(
                    ring_a.at[pl.ds(((p6 - 2) % _SLOTS) * _CHA, _CHA)],
                    out_hbm.at[row, pl.ds(off + c * _CHA, _CHA)],
                    sa_out,
                )

            @pl.when((r >= _SLOTS) & (r < nb + _SLOTS))
            def _():
                pltpu.make_async_copy(
                    ring_b.at[sid, pl.ds(0, _CHB)],
                    out_hbm.at[row, pl.ds(off, _CHB)],
                    sb_out,
                ).wait()

            @pl.when((r >= 2) & (r < nb + 2))
            def _():
                c = r - 2
                pltpu.make_async_copy(
                    table_hbm.at[pl.ds(0, _CHB)],
                    ring_b.at[sid, pl.ds(0, _CHB)],
                    sb[(p6 - 2) % _SLOTS],
                ).wait()
                pltpu.async_copy(
                    ring_b.at[sid, pl.ds(((p6 - 2) % _SLOTS) * _CHB, _CHB)],
                    out_hbm.at[row, pl.ds(off + rows_a + c * _CHB, _CHB)],
                    sb_out,
                )

            @pl.when(r < nb)
            def _():
                half = 8 * (p6 % 2)
                vec = idx_v[pl.ds(rows_a + (i * 3 + p6 // 2) * 16, 16)]
                for j in range(_CHB):
                    pltpu.async_copy(
                        table_hbm.at[vec[half + j]],
                        ring_b.at[sid, sl * _CHB + j],
                        sb[sl],
                    )

        def body(i, carry):
            for p6 in range(6):
                round_(i, i * 6 + p6, p6)
            return carry

        lax.fori_loop(0, n_rounds // 6, body, 0)

    return _gather(idx, table)


def kernel(position_ids, weight):
    batch, seq = position_ids.shape
    vocab, hidden = weight.shape
    return _gather_call(batch, seq, hidden, position_ids, weight)


# speedup vs baseline: 1.0458x; 1.0041x over previous
import functools

import jax
import jax.numpy as jnp
from jax import lax
from jax.experimental import pallas as pl
from jax.experimental.pallas import tpu as pltpu
from jax.experimental.pallas import tpu_sc as plsc

_NUM_CORES = 2
_NUM_SUBCORES = 16
_NUM_WORKERS = _NUM_CORES * _NUM_SUBCORES
_CHUNK = 16
_SLOTS = 3


def _gather_call(batch, seq, hidden, idx, table):
    total = batch * seq
    b_per_w = total // _NUM_WORKERS
    w_per_row = seq // b_per_w
    n_chunks = b_per_w // _CHUNK
    n_rounds = n_chunks + 2
    n_rounds += (-n_rounds) % _SLOTS
    mesh = plsc.VectorSubcoreMesh(core_axis_name="c", subcore_axis_name="s")

    @functools.partial(
        pl.kernel,
        mesh=mesh,
        out_type=jax.ShapeDtypeStruct((batch, seq, hidden), jnp.float32),
        scratch_types=[
            pltpu.VMEM((b_per_w,), jnp.int32),
            pltpu.VMEM_SHARED(
                (_NUM_SUBCORES, _SLOTS * _CHUNK, hidden), jnp.float32
            ),
            pltpu.SemaphoreType.DMA,
            pltpu.SemaphoreType.DMA,
            pltpu.SemaphoreType.DMA,
            pltpu.SemaphoreType.DMA,
        ],
    )
    def _gather(idx_hbm, table_hbm, out_hbm, idx_v, ring, s0, s1, s2, s_out):
        cid = lax.axis_index("c")
        sid = lax.axis_index("s")
        wid = sid * _NUM_CORES + cid
        row = wid // w_per_row
        off = (wid % w_per_row) * b_per_w
        pltpu.sync_copy(idx_hbm.at[row, pl.ds(off, b_per_w)], idx_v)
        sg = (s0, s1, s2)

        def round_(r, p):
            @pl.when((r >= _SLOTS) & (r < n_chunks + _SLOTS))
            def _():
                pltpu.make_async_copy(
                    ring.at[sid, pl.ds(0, _CHUNK)],
                    out_hbm.at[row, pl.ds(off, _CHUNK)],
                    s_out,
                ).wait()

            @pl.when((r >= 2) & (r < n_chunks + 2))
            def _():
                c = r - 2
                pltpu.make_async_copy(
                    table_hbm.at[pl.ds(0, _CHUNK)],
                    ring.at[sid, pl.ds(0, _CHUNK)],
                    sg[(p - 2) % _SLOTS],
                ).wait()
                pltpu.async_copy(
                    ring.at[sid, pl.ds(((p - 2) % _SLOTS) * _CHUNK, _CHUNK)],
                    out_hbm.at[row, pl.ds(off + c * _CHUNK, _CHUNK)],
                    s_out,
                )

            @pl.when(r < n_chunks)
            def _():
                vec = idx_v[pl.ds(r * _CHUNK, _CHUNK)]
                for j in range(_CHUNK):
                    pltpu.async_copy(
                        table_hbm.at[vec[j]],
                        ring.at[sid, p * _CHUNK + j],
                        sg[p],
                    )

        def body(i, carry):
            for p in range(_SLOTS):
                round_(i * _SLOTS + p, p)
            return carry

        lax.fori_loop(0, n_rounds // _SLOTS, body, 0)

    return _gather(idx, table)


def kernel(position_ids, weight):
    batch, seq = position_ids.shape
    vocab, hidden = weight.shape
    return _gather_call(batch, seq, hidden, position_ids, weight)
